# Initial kernel scaffold; baseline (speedup 1.0000x reference)
import jax
import jax.numpy as jnp
from jax.experimental import pallas as pl


def kernel(input, weight):
    raise NotImplementedError("write your pallas kernel here")


# speedup vs baseline: 1.1083x; 1.1083x over previous
import functools

import jax
import jax.numpy as jnp
from jax import lax
from jax.experimental import pallas as pl
from jax.experimental.pallas import tpu as pltpu
from jax.experimental.pallas import tpu_sc as plsc

_NUM_CORES = 2
_NUM_SUBCORES = 16
_NUM_WORKERS = _NUM_CORES * _NUM_SUBCORES
_CHUNK = 2048


@functools.lru_cache(maxsize=None)
def _gather_call(B, D):
    b_per_w = B // _NUM_WORKERS
    n_chunks = b_per_w // _CHUNK
    mesh = plsc.VectorSubcoreMesh(core_axis_name="c", subcore_axis_name="s")

    @functools.partial(
        pl.kernel,
        out_type=jax.ShapeDtypeStruct((B, D), jnp.float32),
        mesh=mesh,
        scratch_types=[
            pltpu.VMEM((_CHUNK,), jnp.int32),
            pltpu.VMEM((_CHUNK, D), jnp.float32),
            pltpu.SemaphoreType.DMA,
        ],
        compiler_params=pltpu.CompilerParams(use_tc_tiling_on_sc=False),
    )
    def body(idx_hbm, table_hbm, out_hbm, idx_v, rows_v, sem):
        wid = lax.axis_index("s") * _NUM_CORES + lax.axis_index("c")
        base = wid * b_per_w

        @pl.loop(0, n_chunks)
        def _chunk(i):
            off = base + i * _CHUNK
            pltpu.sync_copy(idx_hbm.at[pl.ds(off, _CHUNK)], idx_v)
            pltpu.async_copy(table_hbm.at[idx_v], rows_v, sem).wait()
            pltpu.sync_copy(rows_v, out_hbm.at[pl.ds(off, _CHUNK)])

    return body


def kernel(input, weight):
    S0, S1 = input.shape
    B = S0 * S1
    D = weight.shape[1]
    flat_idx = input.reshape(B)
    out = _gather_call(B, D)(flat_idx, weight)
    return out.reshape(S0, S1, D)


# speedup vs baseline: 1.1098x; 1.0014x over previous
import functools

import jax
import jax.numpy as jnp
from jax import lax
from jax.experimental import pallas as pl
from jax.experimental.pallas import tpu as pltpu
from jax.experimental.pallas import tpu_sc as plsc

_NUM_CORES = 2
_NUM_SUBCORES = 16
_NUM_WORKERS = _NUM_CORES * _NUM_SUBCORES
_CHUNK = 1600


@functools.lru_cache(maxsize=None)
def _gather_call(B, D):
    b_per_w = B // _NUM_WORKERS
    n_chunks = b_per_w // _CHUNK
    n_pairs = n_chunks // 2
    mesh = plsc.VectorSubcoreMesh(core_axis_name="c", subcore_axis_name="s")

    @functools.partial(
        pl.kernel,
        out_type=jax.ShapeDtypeStruct((B, D), jnp.float32),
        mesh=mesh,
        scratch_types=[
            pltpu.VMEM((_CHUNK,), jnp.int32),
            pltpu.VMEM((_CHUNK,), jnp.int32),
            pltpu.VMEM((_CHUNK, D), jnp.float32),
            pltpu.VMEM((_CHUNK, D), jnp.float32),
            pltpu.SemaphoreType.DMA,
            pltpu.SemaphoreType.DMA,
        ],
        compiler_params=pltpu.CompilerParams(use_tc_tiling_on_sc=False),
    )
    def body(idx_hbm, table_hbm, out_hbm, idx0, idx1, rows0, rows1, sem0, sem1):
        wid = lax.axis_index("s") * _NUM_CORES + lax.axis_index("c")
        base = wid * b_per_w

        def start_gather(idx_v, rows_v, sem, chunk):
            off = base + chunk * _CHUNK
            pltpu.sync_copy(idx_hbm.at[pl.ds(off, _CHUNK)], idx_v)
            pltpu.async_copy(table_hbm.at[idx_v], rows_v, sem)

        def finish(idx_v, rows_v, sem, chunk):
            off = base + chunk * _CHUNK
            pltpu.make_async_copy(table_hbm.at[idx_v], rows_v, sem).wait()
            pltpu.sync_copy(rows_v, out_hbm.at[pl.ds(off, _CHUNK)])

        start_gather(idx0, rows0, sem0, 0)

        @pl.loop(0, n_pairs)
        def _pair(p):
            k = p * 2
            start_gather(idx1, rows1, sem1, k + 1)
            finish(idx0, rows0, sem0, k)

            @pl.when(p + 1 < n_pairs)
            def _prefetch():
                start_gather(idx0, rows0, sem0, k + 2)

            finish(idx1, rows1, sem1, k + 1)

    return body


def kernel(input, weight):
    S0, S1 = input.shape
    B = S0 * S1
    D = weight.shape[1]
    flat_idx = input.reshape(B)
    out = _gather_call(B, D)(flat_idx, weight)
    return out.reshape(S0, S1, D)


# speedup vs baseline: 1.1105x; 1.0006x over previous
import functools

import jax
import jax.numpy as jnp
from jax import lax
from jax.experimental import pallas as pl
from jax.experimental.pallas import tpu as pltpu
from jax.experimental.pallas import tpu_sc as plsc

_NUM_CORES = 2
_NUM_SUBCORES = 16
_NUM_WORKERS = _NUM_CORES * _NUM_SUBCORES
_CHUNK = 1600
_STREAMS = 8
_SUB = _CHUNK // _STREAMS


@functools.lru_cache(maxsize=None)
def _gather_call(B, D):
    b_per_w = B // _NUM_WORKERS
    n_chunks = b_per_w // _CHUNK
    n_pairs = n_chunks // 2
    mesh = plsc.VectorSubcoreMesh(core_axis_name="c", subcore_axis_name="s")

    @functools.partial(
        pl.kernel,
        out_type=jax.ShapeDtypeStruct((B, D), jnp.float32),
        mesh=mesh,
        scratch_types=[
            pltpu.VMEM((_CHUNK,), jnp.int32),
            pltpu.VMEM((_CHUNK,), jnp.int32),
            pltpu.VMEM((_CHUNK, D), jnp.float32),
            pltpu.VMEM((_CHUNK, D), jnp.float32),
            pltpu.SemaphoreType.DMA,
            pltpu.SemaphoreType.DMA,
        ],
        compiler_params=pltpu.CompilerParams(use_tc_tiling_on_sc=False),
    )
    def body(idx_hbm, table_hbm, out_hbm, idx0, idx1, rows0, rows1, sem0, sem1):
        wid = lax.axis_index("s") * _NUM_CORES + lax.axis_index("c")
        base = wid * b_per_w

        def start_gather(idx_v, rows_v, sem, chunk):
            off = base + chunk * _CHUNK
            pltpu.sync_copy(idx_hbm.at[pl.ds(off, _CHUNK)], idx_v)
            for j in range(_STREAMS):
                s = j * _SUB
                pltpu.async_copy(
                    table_hbm.at[idx_v.at[pl.ds(s, _SUB)]],
                    rows_v.at[pl.ds(s, _SUB)],
                    sem,
                )

        def finish(idx_v, rows_v, sem, chunk):
            off = base + chunk * _CHUNK
            pltpu.make_async_copy(table_hbm.at[idx_v], rows_v, sem).wait()
            pltpu.sync_copy(rows_v, out_hbm.at[pl.ds(off, _CHUNK)])

        start_gather(idx0, rows0, sem0, 0)

        @pl.loop(0, n_pairs)
        def _pair(p):
            k = p * 2
            start_gather(idx1, rows1, sem1, k + 1)
            finish(idx0, rows0, sem0, k)

            @pl.when(p + 1 < n_pairs)
            def _prefetch():
                start_gather(idx0, rows0, sem0, k + 2)

            finish(idx1, rows1, sem1, k + 1)

    return body


def kernel(input, weight):
    S0, S1 = input.shape
    B = S0 * S1
    D = weight.shape[1]
    flat_idx = input.reshape(B)
    out = _gather_call(B, D)(flat_idx, weight)
    return out.reshape(S0, S1, D)


# speedup vs baseline: 3.5105x; 3.1612x over previous
import functools

import jax
import jax.numpy as jnp
from jax import lax
from jax.experimental import pallas as pl
from jax.experimental.pallas import tpu as pltpu
from jax.experimental.pallas import tpu_sc as plsc

_NUM_CORES = 2
_NUM_SUBCORES = 16
_NUM_WORKERS = _NUM_CORES * _NUM_SUBCORES
_BBLK = 1024
_L = 16


@functools.lru_cache(maxsize=None)
def _gather_call(NB, NS, V, D):
    assert D == 32 and NB % _BBLK == 0
    n_units = NS * (NB // _BBLK)
    units_per_w = n_units // _NUM_WORKERS
    assert units_per_w * _NUM_WORKERS == n_units
    blk_per_s = NB // _BBLK
    DT, DI = D // 8, 8
    BT = _BBLK // 128
    mesh = plsc.VectorSubcoreMesh(core_axis_name="c", subcore_axis_name="s")

    @functools.partial(
        pl.kernel,
        out_type=jax.ShapeDtypeStruct((NS, DT, NB // 128, DI, 128), jnp.float32),
        mesh=mesh,
        scratch_types=[
            pltpu.VMEM((_BBLK,), jnp.int32),
            pltpu.VMEM((_BBLK, D), jnp.float32),
            pltpu.VMEM((DT, BT, DI, 128), jnp.float32),
            pltpu.SemaphoreType.DMA,
        ],
        compiler_params=pltpu.CompilerParams(
            use_tc_tiling_on_sc=False, needs_layout_passes=False
        ),
    )
    def body(idx_hbm, table_hbm, out_hbm, idx_v, rows_v, t_v, sem):
        wid = lax.axis_index("s") * _NUM_CORES + lax.axis_index("c")
        lane_rows = lax.iota(jnp.int32, 16)

        @pl.loop(0, units_per_w)
        def _unit(j):
            u = wid * units_per_w + j
            s = u // blk_per_s
            blk = u % blk_per_s
            off = s * NB + blk * _BBLK
            pltpu.sync_copy(idx_hbm.at[pl.ds(off, _BBLK)], idx_v)
            pltpu.async_copy(table_hbm.at[idx_v], rows_v, sem).wait()

            @pl.loop(0, BT * DI)
            def _tile(q):
                bt = q // DI
                di = q % DI
                row0 = bt * 128
                for dt in range(DT):
                    d = dt * 8 + di
                    for bic in range(8):
                        rows = row0 + bic * 16 + lane_rows
                        cols = jnp.full((16,), d, dtype=jnp.int32)
                        vec = plsc.load_gather(rows_v, [rows, cols])
                        t_v[dt, bt, di, pl.ds(bic * 16, 16)] = vec

            for dt in range(DT):
                pltpu.sync_copy(
                    t_v.at[dt], out_hbm.at[s, dt, pl.ds(blk * BT, BT)]
                )

    return body


def kernel(input, weight):
    NB, NS = input.shape
    V, D = weight.shape
    flat_idx = input.T.reshape(NB * NS)
    a5 = _gather_call(NB, NS, V, D)(flat_idx, weight)
    return a5.transpose(2, 4, 0, 1, 3).reshape(NB, NS, D)


# speedup vs baseline: 8.3744x; 2.3855x over previous
import functools

import jax
import jax.numpy as jnp
from jax import lax
from jax.experimental import pallas as pl
from jax.experimental.pallas import tpu as pltpu
from jax.experimental.pallas import tpu_sc as plsc

_NUM_CORES = 2
_NUM_SUBCORES = 16
_NUM_WORKERS = _NUM_CORES * _NUM_SUBCORES
_BBLK = 1024
_L = 16


@functools.lru_cache(maxsize=None)
def _gather_call(NB, NS, V, D):
    assert D == 32 and NB % _BBLK == 0
    n_units = NS * (NB // _BBLK)
    units_per_w = n_units // _NUM_WORKERS
    assert units_per_w * _NUM_WORKERS == n_units
    blk_per_s = NB // _BBLK
    DT, DI = D // 8, 8
    BT = _BBLK // 128
    mesh = plsc.VectorSubcoreMesh(core_axis_name="c", subcore_axis_name="s")

    @functools.partial(
        pl.kernel,
        out_type=jax.ShapeDtypeStruct((NS, DT, NB // 128, DI, 128), jnp.float32),
        mesh=mesh,
        scratch_types=[
            pltpu.VMEM((_BBLK,), jnp.int32),
            pltpu.VMEM((_BBLK, D), jnp.float32),
            pltpu.VMEM((DT, BT, DI, 129), jnp.float32),
            pltpu.SemaphoreType.DMA,
        ],
        compiler_params=pltpu.CompilerParams(
            use_tc_tiling_on_sc=False, needs_layout_passes=False
        ),
    )
    def body(idx_hbm, table_hbm, out_hbm, idx_v, rows_v, t_v, sem):
        wid = lax.axis_index("s") * _NUM_CORES + lax.axis_index("c")
        lanes = lax.iota(jnp.int32, 16)
        dtv = [2 * h + lanes // 8 for h in (0, 1)]
        div = lanes % 8

        @pl.loop(0, units_per_w)
        def _unit(j):
            u = wid * units_per_w + j
            s = u // blk_per_s
            blk = u % blk_per_s
            off = s * NB + blk * _BBLK
            pltpu.sync_copy(idx_hbm.at[pl.ds(off, _BBLK)], idx_v)
            pltpu.async_copy(table_hbm.at[idx_v], rows_v, sem).wait()

            @plsc.parallel_loop(0, _BBLK, step=8, unroll=4)
            def _rows(r0):
                for rr in range(8):
                    r = r0 + rr
                    btv = jnp.full((16,), r // 128, dtype=jnp.int32)
                    biv = jnp.full((16,), r % 128, dtype=jnp.int32)
                    for h in (0, 1):
                        vec = rows_v[r, pl.ds(16 * h, 16)]
                        plsc.store_scatter(t_v, [dtv[h], btv, div, biv], vec)

            for dt in range(DT):
                pltpu.sync_copy(
                    t_v.at[dt, :, :, pl.ds(0, 128)],
                    out_hbm.at[s, dt, pl.ds(blk * BT, BT)],
                )

    return body


def kernel(input, weight):
    NB, NS = input.shape
    V, D = weight.shape
    flat_idx = input.T.reshape(NB * NS)
    a5 = _gather_call(NB, NS, V, D)(flat_idx, weight)
    return a5.transpose(2, 4, 0, 1, 3).reshape(NB, NS, D)


# speedup vs baseline: 9.0007x; 1.0748x over previous
import functools

import jax
import jax.numpy as jnp
from jax import lax
from jax.experimental import pallas as pl
from jax.experimental.pallas import tpu as pltpu
from jax.experimental.pallas import tpu_sc as plsc

_NUM_CORES = 2
_NUM_SUBCORES = 16
_NUM_WORKERS = _NUM_CORES * _NUM_SUBCORES
_BBLK = 512
_L = 16


@functools.lru_cache(maxsize=None)
def _gather_call(NB, NS, V, D):
    assert D == 32 and NB % _BBLK == 0
    n_units = NS * (NB // _BBLK)
    units_per_w = n_units // _NUM_WORKERS
    assert units_per_w * _NUM_WORKERS == n_units
    blk_per_s = NB // _BBLK
    DT, DI = D // 8, 8
    BT = _BBLK // 128
    mesh = plsc.VectorSubcoreMesh(core_axis_name="c", subcore_axis_name="s")

    @functools.partial(
        pl.kernel,
        out_type=jax.ShapeDtypeStruct((NS, DT, NB // 128, DI, 128), jnp.float32),
        mesh=mesh,
        scratch_types=[
            pltpu.VMEM((_BBLK,), jnp.int32),
            pltpu.VMEM((_BBLK,), jnp.int32),
            pltpu.VMEM((_BBLK, D), jnp.float32),
            pltpu.VMEM((_BBLK, D), jnp.float32),
            pltpu.VMEM((DT, BT, DI, 129), jnp.float32),
            pltpu.VMEM((DT, BT, DI, 129), jnp.float32),
            pltpu.SemaphoreType.DMA,
            pltpu.SemaphoreType.DMA,
        ],
        compiler_params=pltpu.CompilerParams(
            use_tc_tiling_on_sc=False, needs_layout_passes=False
        ),
    )
    def body(
        idx_hbm, table_hbm, out_hbm, idx0, idx1, rows0, rows1, t0, t1, sem0, sem1
    ):
        wid = lax.axis_index("s") * _NUM_CORES + lax.axis_index("c")
        lanes = lax.iota(jnp.int32, 16)
        dtv = [2 * h + lanes // 8 for h in (0, 1)]
        div = lanes % 8
        bufs = ((idx0, rows0, t0, sem0), (idx1, rows1, t1, sem1))

        def start(b, u):
            idx_v, rows_v, _, sem = bufs[b]
            s = u // blk_per_s
            blk = u % blk_per_s
            pltpu.sync_copy(
                idx_hbm.at[pl.ds(s * NB + blk * _BBLK, _BBLK)], idx_v
            )
            pltpu.async_copy(table_hbm.at[idx_v], rows_v, sem)

        def finish(b, u):
            idx_v, rows_v, t_v, sem = bufs[b]
            s = u // blk_per_s
            blk = u % blk_per_s
            pltpu.make_async_copy(table_hbm.at[idx_v], rows_v, sem).wait()

            @plsc.parallel_loop(0, _BBLK, step=8, unroll=4)
            def _rows(r0):
                for rr in range(8):
                    r = r0 + rr
                    btv = jnp.full((16,), r // 128, dtype=jnp.int32)
                    biv = jnp.full((16,), r % 128, dtype=jnp.int32)
                    for h in (0, 1):
                        vec = rows_v[r, pl.ds(16 * h, 16)]
                        plsc.store_scatter(t_v, [dtv[h], btv, div, biv], vec)

            for dt in range(DT):
                pltpu.sync_copy(
                    t_v.at[dt, :, :, pl.ds(0, 128)],
                    out_hbm.at[s, dt, pl.ds(blk * BT, BT)],
                )

        u0 = wid * units_per_w
        start(0, u0)

        @pl.loop(0, units_per_w // 2)
        def _pair(p):
            k = u0 + 2 * p
            start(1, k + 1)
            finish(0, k)

            @pl.when(2 * p + 2 < units_per_w)
            def _prefetch():
                start(0, k + 2)

            finish(1, k + 1)

    return body


def kernel(input, weight):
    NB, NS = input.shape
    V, D = weight.shape
    flat_idx = input.T.reshape(NB * NS)
    a5 = _gather_call(NB, NS, V, D)(flat_idx, weight)
    return a5.transpose(2, 4, 0, 1, 3).reshape(NB, NS, D)
